# baseline (speedup 1.0000x reference)
import functools

import jax
import jax.numpy as jnp
from jax import lax
from jax.experimental import pallas as pl
from jax.experimental.pallas import tpu as pltpu
from jax.experimental.pallas import tpu_sc as plsc

N = 10000
E = 320000
D = 128
H = 64
NT = 16
NCORE = 2
LANE = 128
R = 160
EP = NT * R * LANE
K = 4
KD = 8
RD = R // NCORE
NP_ = NT * 640
N_DEG = NP_
ROWS_T = NP_ // NT
BLK = 1280
GRID = NP_ // BLK

_mesh = plsc.VectorSubcoreMesh(core_axis_name="c", subcore_axis_name="s")
_sc_params = pltpu.CompilerParams(use_tc_tiling_on_sc=False)


@functools.partial(
    pl.kernel,
    out_type=jax.ShapeDtypeStruct((NCORE, N_DEG), jnp.float32),
    mesh=_mesh,
    compiler_params=_sc_params,
    scratch_types=[
        pltpu.VMEM_SHARED((N_DEG,), jnp.float32),
        pltpu.VMEM((KD, LANE), jnp.int32),
        pltpu.VMEM((KD, LANE), jnp.float32),
        pltpu.VMEM((640,), jnp.float32),
    ],
)
def _deg_kernel(dst_hbm, w_hbm, deg_hbm, deg_sp, dst_v, w_v, zz):
    c = lax.axis_index("c")
    s = lax.axis_index("s")
    for i in range(640 // 16):
        zz[pl.ds(i * 16, 16)] = jnp.zeros((16,), jnp.float32)
    pltpu.sync_copy(zz, deg_sp.at[pl.ds(s * 640, 640)])
    plsc.subcore_barrier()

    def chunk(i, carry):
        r0 = c * RD + i * KD
        pltpu.sync_copy(dst_hbm.at[s].at[pl.ds(r0, KD)], dst_v)
        pltpu.sync_copy(w_hbm.at[s].at[pl.ds(r0, KD)], w_v)
        for j in range(KD):
            pltpu.sync_copy(w_v.at[j], deg_sp.at[dst_v.at[j]], add=True)
        return carry

    lax.fori_loop(0, RD // KD, chunk, 0)
    plsc.subcore_barrier()
    pltpu.sync_copy(deg_sp.at[pl.ds(s * 640, 640)],
                    deg_hbm.at[c].at[pl.ds(s * 640, 640)])


KM = 2
NCHUNK = R // KM

def _make_msg_kernel(final):
    out_shape = (jax.ShapeDtypeStruct((N, D), jnp.float32) if final
                 else jax.ShapeDtypeStruct((NP_, D), jnp.float32))
    extra_scratch = ([pltpu.VMEM((ROWS_T,), jnp.float32),
                      pltpu.VMEM((H,), jnp.float32)] if final else [])

    def body(*refs):
        if final:
            (xwp_hbm, src_hbm, dst_hbm, w_hbm, dinv_hbm, b_hbm, out_hbm,
             xw_sp, agg_sp, src_v, dst_v, w_v, rows_v,
             gsem, ssem, isem, lsem, dinv_v, bias_v) = refs
        else:
            (xwp_hbm, src_hbm, dst_hbm, w_hbm, out_hbm,
             xw_sp, agg_sp, src_v, dst_v, w_v, rows_v,
             gsem, ssem, isem, lsem) = refs
        c = lax.axis_index("c")
        s = lax.axis_index("s")
        n0 = s * ROWS_T
        half = xwp_hbm.at[pl.ds(n0, ROWS_T), pl.ds(c * H, H)]
        loads = [
            pltpu.make_async_copy(half, xw_sp.at[pl.ds(n0, ROWS_T)], lsem),
            pltpu.make_async_copy(half, agg_sp.at[pl.ds(n0, ROWS_T)], lsem),
        ]
        if final:
            loads.append(pltpu.make_async_copy(
                dinv_hbm.at[pl.ds(n0, ROWS_T)], dinv_v, lsem))
            loads.append(pltpu.make_async_copy(
                b_hbm.at[pl.ds(c * H, H)], bias_v, lsem))
        for cp in loads:
            cp.start()

        def idx_copies(i, bi):
            r0 = i * KM
            return [
                pltpu.make_async_copy(src_hbm.at[s].at[pl.ds(r0, KM)],
                                      src_v.at[bi], isem.at[bi]),
                pltpu.make_async_copy(dst_hbm.at[s].at[pl.ds(r0, KM)],
                                      dst_v.at[bi], isem.at[bi]),
                pltpu.make_async_copy(w_hbm.at[s].at[pl.ds(r0, KM)],
                                      w_v.at[bi], isem.at[bi]),
            ]

        for cp in idx_copies(0, 0) + idx_copies(1, 1):
            cp.start()

        for cp in loads:
            cp.wait()
        plsc.subcore_barrier()

        def gathers(i, bi, br):
            return [pltpu.make_async_copy(
                xw_sp.at[src_v.at[bi].at[j]],
                rows_v.at[br].at[pl.ds(j * LANE, LANE)], gsem.at[br])
                for j in range(KM)]

        def scatters(i, bi, br):
            return [pltpu.make_async_copy(
                rows_v.at[br].at[pl.ds(j * LANE, LANE)],
                agg_sp.at[dst_v.at[bi].at[j]], ssem.at[br])
                for j in range(KM)]

        def fire_scatters(i, bi, br):
            for j in range(KM):
                pltpu.async_copy(
                    rows_v.at[br].at[pl.ds(j * LANE, LANE)],
                    agg_sp.at[dst_v.at[bi].at[j]], ssem.at[br], add=True)

        def scale_chunk(i, bi, br):
            for j in range(KM):
                def _scale(g, j=j):
                    w16 = w_v[bi, j, pl.ds(g, 16)]
                    for l in range(16):
                        wv = w16[l]
                        e = j * LANE + g + l
                        for f in range(H // 16):
                            rows_v[br, e, pl.ds(f * 16, 16)] = (
                                rows_v[br, e, pl.ds(f * 16, 16)] * wv)

                plsc.parallel_loop(0, LANE, step=16, unroll=4)(_scale)

        def process_prev(i, bi, br):
            for cp in gathers(i, bi, br):
                cp.wait()
            scale_chunk(i, bi, br)
            fire_scatters(i, bi, br)

        def outer(i2, carry):
            for b in range(4):
                i = i2 * 4 + b
                bi = b
                br = b % 2
                for cp in idx_copies(i, bi):
                    cp.wait()

                if b < 2:
                    @pl.when(i2 >= 1)
                    def _():
                        for cp in scatters(i - 2, (b + 2) % 4, br):
                            cp.wait()
                else:
                    for cp in scatters(i - 2, (b + 2) % 4, br):
                        cp.wait()

                for cp in gathers(i, bi, br):
                    cp.start()

                @pl.when(i < NCHUNK - 2)
                def _(i=i, b=b):
                    for cp in idx_copies(i + 2, (b + 2) % 4):
                        cp.start()

                if b == 0:
                    @pl.when(i2 >= 1)
                    def _():
                        process_prev(i - 1, 3, 1)
                else:
                    process_prev(i - 1, b - 1, (b - 1) % 2)
            return carry

        lax.fori_loop(0, NCHUNK // 4, outer, 0)
        process_prev(NCHUNK - 1, 3, 1)
        for cp in scatters(NCHUNK - 2, 2, 0):
            cp.wait()
        for cp in scatters(NCHUNK - 1, 3, 1):
            cp.wait()
        plsc.subcore_barrier()

        if not final:
            pltpu.sync_copy(agg_sp.at[pl.ds(n0, ROWS_T)],
                            out_hbm.at[pl.ds(n0, ROWS_T), pl.ds(c * H, H)])
            return

        b4 = [bias_v[pl.ds(f * 16, 16)] for f in range(H // 16)]
        WB = 80

        def _wb_slab(q, carry):
            off = q * WB

            @pl.when(n0 + off < N)
            def _():
                pltpu.sync_copy(agg_sp.at[pl.ds(n0 + off, WB)],
                                rows_v.at[0].at[pl.ds(0, WB)])

                def _wb(g):
                    d16 = dinv_v[pl.ds(off + g, 16)]
                    for l in range(16):
                        dv = d16[l]
                        for f in range(H // 16):
                            rows_v[0, g + l, pl.ds(f * 16, 16)] = (
                                rows_v[0, g + l, pl.ds(f * 16, 16)] * dv
                                + b4[f])

                plsc.parallel_loop(0, WB, step=16, unroll=2)(_wb)
                pltpu.sync_copy(
                    rows_v.at[0].at[pl.ds(0, WB)],
                    out_hbm.at[pl.ds(n0 + off, WB), pl.ds(c * H, H)])

            return carry

        lax.fori_loop(0, ROWS_T // WB, _wb_slab, 0)

    return functools.partial(
        pl.kernel,
        out_type=out_shape,
        mesh=_mesh,
        compiler_params=_sc_params,
        scratch_types=[
            pltpu.VMEM_SHARED((NP_, H), jnp.float32),
            pltpu.VMEM_SHARED((NP_, H), jnp.float32),
            pltpu.VMEM((4, KM, LANE), jnp.int32),
            pltpu.VMEM((4, KM, LANE), jnp.int32),
            pltpu.VMEM((4, KM, LANE), jnp.float32),
            pltpu.VMEM((2, KM * LANE, H), jnp.float32),
            pltpu.SemaphoreType.DMA((2,)),
            pltpu.SemaphoreType.DMA((2,)),
            pltpu.SemaphoreType.DMA((4,)),
            pltpu.SemaphoreType.DMA,
        ] + extra_scratch,
    )(body)


_msg_kernel = _make_msg_kernel(final=False)
_msg_final_kernel = _make_msg_kernel(final=True)


def _mm1_body(x_ref, w1_ref, deg_ref, xwp_ref, dinv_ref):
    deg = deg_ref[0] + deg_ref[1] + 1.0
    dinv = lax.rsqrt(jnp.maximum(deg, 1e-12))
    t = jnp.dot(x_ref[...], w1_ref[...], preferred_element_type=jnp.float32)
    xwp_ref[...] = t * dinv
    dinv_ref[...] = dinv


def _mm2_body(agg_ref, dinv_ref, b1_ref, w2_ref, xwp2_ref):
    dinv = dinv_ref[...]
    h = jnp.maximum(agg_ref[...] * dinv + b1_ref[...], 0.0)
    t = jnp.dot(h, w2_ref[...], preferred_element_type=jnp.float32)
    xwp2_ref[...] = t * dinv


def _final_body(agg_ref, dinv_ref, b2_ref, out_ref):
    out_ref[...] = agg_ref[...] * dinv_ref[...] + b2_ref[...]


def kernel(x, edge_index, edge_weight, W1, b1, W2, b2):
    src = edge_index[0].astype(jnp.int32)
    dst = edge_index[1].astype(jnp.int32)
    x_p = jnp.pad(x, ((0, NP_ - N), (0, 0)))
    pad = EP - E
    pad_idx = jnp.arange(pad, dtype=jnp.int32) % N
    src_p = jnp.concatenate([src, pad_idx]).reshape(NT, R, LANE)
    dst_p = jnp.concatenate([dst, pad_idx]).reshape(NT, R, LANE)
    w_p = jnp.concatenate(
        [edge_weight, jnp.zeros((pad,), jnp.float32)]).reshape(NT, R, LANE)

    deg_parts = _deg_kernel(dst_p, w_p)
    deg2 = deg_parts.reshape(NCORE, NP_, 1)

    b1r = b1.reshape(1, D)
    b2r = b2.reshape(1, D)

    xwp1, dinv = pl.pallas_call(
        _mm1_body,
        grid=(GRID,),
        in_specs=[
            pl.BlockSpec((BLK, D), lambda i: (i, 0)),
            pl.BlockSpec((D, D), lambda i: (0, 0)),
            pl.BlockSpec((NCORE, BLK, 1), lambda i: (0, i, 0)),
        ],
        out_specs=[
            pl.BlockSpec((BLK, D), lambda i: (i, 0)),
            pl.BlockSpec((BLK, 1), lambda i: (i, 0)),
        ],
        out_shape=[
            jax.ShapeDtypeStruct((NP_, D), jnp.float32),
            jax.ShapeDtypeStruct((NP_, 1), jnp.float32),
        ],
    )(x_p, W1, deg2)

    agg1 = _msg_kernel(xwp1, src_p, dst_p, w_p)

    xwp2 = pl.pallas_call(
        _mm2_body,
        grid=(GRID,),
        in_specs=[
            pl.BlockSpec((BLK, D), lambda i: (i, 0)),
            pl.BlockSpec((BLK, 1), lambda i: (i, 0)),
            pl.BlockSpec((1, D), lambda i: (0, 0)),
            pl.BlockSpec((D, D), lambda i: (0, 0)),
        ],
        out_specs=pl.BlockSpec((BLK, D), lambda i: (i, 0)),
        out_shape=jax.ShapeDtypeStruct((NP_, D), jnp.float32),
    )(agg1, dinv, b1r, W2)

    out = _msg_final_kernel(xwp2, src_p, dst_p, w_p,
                            dinv.reshape(NP_), b2)
    return out


# speedup vs baseline: 1.1445x; 1.1445x over previous
import functools

import jax
import jax.numpy as jnp
from jax import lax
from jax.experimental import pallas as pl
from jax.experimental.pallas import tpu as pltpu
from jax.experimental.pallas import tpu_sc as plsc

N = 10000
E = 320000
D = 128
H = 64
NT = 16
NCORE = 2
LANE = 128
R = 160
EP = NT * R * LANE
K = 4
KD = 8
RD = R // NCORE
NP_ = NT * 640
N_DEG = NP_
ROWS_T = NP_ // NT
BLK = 1280
GRID = NP_ // BLK

_mesh = plsc.VectorSubcoreMesh(core_axis_name="c", subcore_axis_name="s")
_sc_params = pltpu.CompilerParams(use_tc_tiling_on_sc=False)


KD = 5
NDCH = RD // KD

@functools.partial(
    pl.kernel,
    out_type=jax.ShapeDtypeStruct((NCORE, N_DEG), jnp.float32),
    mesh=_mesh,
    compiler_params=_sc_params,
    scratch_types=[
        pltpu.VMEM_SHARED((N_DEG,), jnp.float32),
        pltpu.VMEM((4, KD, LANE), jnp.int32),
        pltpu.VMEM((4, KD, LANE), jnp.float32),
        pltpu.VMEM((640,), jnp.float32),
        pltpu.SemaphoreType.DMA((4,)),
        pltpu.SemaphoreType.DMA((2,)),
    ],
)
def _deg_kernel(dst_hbm, w_hbm, deg_hbm, deg_sp, dst_v, w_v, zz, isem, ssem):
    c = lax.axis_index("c")
    s = lax.axis_index("s")
    for i in range(640 // 16):
        zz[pl.ds(i * 16, 16)] = jnp.zeros((16,), jnp.float32)

    def idx_copies(i, bi):
        r0 = c * RD + i * KD
        return [
            pltpu.make_async_copy(dst_hbm.at[s].at[pl.ds(r0, KD)],
                                  dst_v.at[bi], isem.at[bi]),
            pltpu.make_async_copy(w_hbm.at[s].at[pl.ds(r0, KD)],
                                  w_v.at[bi], isem.at[bi]),
        ]

    for cp in idx_copies(0, 0) + idx_copies(1, 1):
        cp.start()
    pltpu.sync_copy(zz, deg_sp.at[pl.ds(s * 640, 640)])
    plsc.subcore_barrier()

    def scatters(bi, br):
        return [pltpu.make_async_copy(w_v.at[bi].at[j],
                                      deg_sp.at[dst_v.at[bi].at[j]],
                                      ssem.at[br])
                for j in range(KD)]

    def outer(i2, carry):
        for b in range(4):
            i = i2 * 4 + b
            br = b % 2
            for cp in idx_copies(i, b):
                cp.wait()
            if b < 2:
                @pl.when(i2 >= 1)
                def _():
                    for cp in scatters((b + 2) % 4, br):
                        cp.wait()
            else:
                for cp in scatters((b + 2) % 4, br):
                    cp.wait()
            for j in range(KD):
                pltpu.async_copy(w_v.at[b].at[j],
                                 deg_sp.at[dst_v.at[b].at[j]],
                                 ssem.at[br], add=True)

            @pl.when(i < NDCH - 2)
            def _(i=i, b=b):
                for cp in idx_copies(i + 2, (b + 2) % 4):
                    cp.start()
        return carry

    lax.fori_loop(0, NDCH // 4, outer, 0)
    for cp in scatters(2, 0) + scatters(3, 1):
        cp.wait()
    plsc.subcore_barrier()
    pltpu.sync_copy(deg_sp.at[pl.ds(s * 640, 640)],
                    deg_hbm.at[c].at[pl.ds(s * 640, 640)])


KM = 2
NCHUNK = R // KM

def _make_msg_kernel(final):
    out_shape = (jax.ShapeDtypeStruct((N, D), jnp.float32) if final
                 else jax.ShapeDtypeStruct((NP_, D), jnp.float32))
    extra_scratch = ([pltpu.VMEM((ROWS_T,), jnp.float32),
                      pltpu.VMEM((H,), jnp.float32)] if final else [])

    def body(*refs):
        if final:
            (xwp_hbm, src_hbm, dst_hbm, w_hbm, dinv_hbm, b_hbm, out_hbm,
             xw_sp, agg_sp, src_v, dst_v, w_v, rows_v,
             gsem, ssem, isem, lsem, dinv_v, bias_v) = refs
        else:
            (xwp_hbm, src_hbm, dst_hbm, w_hbm, out_hbm,
             xw_sp, agg_sp, src_v, dst_v, w_v, rows_v,
             gsem, ssem, isem, lsem) = refs
        c = lax.axis_index("c")
        s = lax.axis_index("s")
        n0 = s * ROWS_T
        half = xwp_hbm.at[pl.ds(n0, ROWS_T), pl.ds(c * H, H)]
        loads = [
            pltpu.make_async_copy(half, xw_sp.at[pl.ds(n0, ROWS_T)], lsem),
            pltpu.make_async_copy(half, agg_sp.at[pl.ds(n0, ROWS_T)], lsem),
        ]
        if final:
            loads.append(pltpu.make_async_copy(
                dinv_hbm.at[pl.ds(n0, ROWS_T)], dinv_v, lsem))
            loads.append(pltpu.make_async_copy(
                b_hbm.at[pl.ds(c * H, H)], bias_v, lsem))
        for cp in loads:
            cp.start()

        def idx_copies(i, bi):
            r0 = i * KM
            return [
                pltpu.make_async_copy(src_hbm.at[s].at[pl.ds(r0, KM)],
                                      src_v.at[bi], isem.at[bi]),
                pltpu.make_async_copy(dst_hbm.at[s].at[pl.ds(r0, KM)],
                                      dst_v.at[bi], isem.at[bi]),
                pltpu.make_async_copy(w_hbm.at[s].at[pl.ds(r0, KM)],
                                      w_v.at[bi], isem.at[bi]),
            ]

        for cp in idx_copies(0, 0) + idx_copies(1, 1):
            cp.start()

        for cp in loads:
            cp.wait()
        plsc.subcore_barrier()

        def gathers(i, bi, br):
            return [pltpu.make_async_copy(
                xw_sp.at[src_v.at[bi].at[j]],
                rows_v.at[br].at[pl.ds(j * LANE, LANE)], gsem.at[br])
                for j in range(KM)]

        def scatters(i, bi, br):
            return [pltpu.make_async_copy(
                rows_v.at[br].at[pl.ds(j * LANE, LANE)],
                agg_sp.at[dst_v.at[bi].at[j]], ssem.at[br])
                for j in range(KM)]

        def fire_scatters(i, bi, br):
            for j in range(KM):
                pltpu.async_copy(
                    rows_v.at[br].at[pl.ds(j * LANE, LANE)],
                    agg_sp.at[dst_v.at[bi].at[j]], ssem.at[br], add=True)

        def scale_chunk(i, bi, br):
            for j in range(KM):
                def _scale(g, j=j):
                    w16 = w_v[bi, j, pl.ds(g, 16)]
                    for l in range(16):
                        wv = w16[l]
                        e = j * LANE + g + l
                        for f in range(H // 16):
                            rows_v[br, e, pl.ds(f * 16, 16)] = (
                                rows_v[br, e, pl.ds(f * 16, 16)] * wv)

                plsc.parallel_loop(0, LANE, step=16, unroll=2)(_scale)

        def process_prev(i, bi, br):
            for cp in gathers(i, bi, br):
                cp.wait()
            scale_chunk(i, bi, br)
            fire_scatters(i, bi, br)

        def outer(i2, carry):
            for b in range(4):
                i = i2 * 4 + b
                bi = b
                br = b % 2
                for cp in idx_copies(i, bi):
                    cp.wait()

                if b < 2:
                    @pl.when(i2 >= 1)
                    def _():
                        for cp in scatters(i - 2, (b + 2) % 4, br):
                            cp.wait()
                else:
                    for cp in scatters(i - 2, (b + 2) % 4, br):
                        cp.wait()

                for cp in gathers(i, bi, br):
                    cp.start()

                @pl.when(i < NCHUNK - 2)
                def _(i=i, b=b):
                    for cp in idx_copies(i + 2, (b + 2) % 4):
                        cp.start()

                if b == 0:
                    @pl.when(i2 >= 1)
                    def _():
                        process_prev(i - 1, 3, 1)
                else:
                    process_prev(i - 1, b - 1, (b - 1) % 2)
            return carry

        lax.fori_loop(0, NCHUNK // 4, outer, 0)
        process_prev(NCHUNK - 1, 3, 1)
        for cp in scatters(NCHUNK - 2, 2, 0):
            cp.wait()
        for cp in scatters(NCHUNK - 1, 3, 1):
            cp.wait()
        plsc.subcore_barrier()

        if not final:
            pltpu.sync_copy(agg_sp.at[pl.ds(n0, ROWS_T)],
                            out_hbm.at[pl.ds(n0, ROWS_T), pl.ds(c * H, H)])
            return

        b4 = [bias_v[pl.ds(f * 16, 16)] for f in range(H // 16)]
        WB = 80

        def _wb_slab(q, carry):
            off = q * WB

            @pl.when(n0 + off < N)
            def _():
                pltpu.sync_copy(agg_sp.at[pl.ds(n0 + off, WB)],
                                rows_v.at[0].at[pl.ds(0, WB)])

                def _wb(g):
                    d16 = dinv_v[pl.ds(off + g, 16)]
                    for l in range(16):
                        dv = d16[l]
                        for f in range(H // 16):
                            rows_v[0, g + l, pl.ds(f * 16, 16)] = (
                                rows_v[0, g + l, pl.ds(f * 16, 16)] * dv
                                + b4[f])

                plsc.parallel_loop(0, WB, step=16)(_wb)
                pltpu.sync_copy(
                    rows_v.at[0].at[pl.ds(0, WB)],
                    out_hbm.at[pl.ds(n0 + off, WB), pl.ds(c * H, H)])

            return carry

        lax.fori_loop(0, ROWS_T // WB, _wb_slab, 0)

    return functools.partial(
        pl.kernel,
        out_type=out_shape,
        mesh=_mesh,
        compiler_params=_sc_params,
        scratch_types=[
            pltpu.VMEM_SHARED((NP_, H), jnp.float32),
            pltpu.VMEM_SHARED((NP_, H), jnp.float32),
            pltpu.VMEM((4, KM, LANE), jnp.int32),
            pltpu.VMEM((4, KM, LANE), jnp.int32),
            pltpu.VMEM((4, KM, LANE), jnp.float32),
            pltpu.VMEM((2, KM * LANE, H), jnp.float32),
            pltpu.SemaphoreType.DMA((2,)),
            pltpu.SemaphoreType.DMA((2,)),
            pltpu.SemaphoreType.DMA((4,)),
            pltpu.SemaphoreType.DMA,
        ] + extra_scratch,
    )(body)


_msg_kernel = _make_msg_kernel(final=False)
_msg_final_kernel = _make_msg_kernel(final=True)


def _mm1_body(x_ref, w1_ref, deg_ref, xwp_ref, dinv_ref):
    deg = deg_ref[0] + deg_ref[1] + 1.0
    dinv = lax.rsqrt(jnp.maximum(deg, 1e-12))
    t = jnp.dot(x_ref[...], w1_ref[...], preferred_element_type=jnp.float32)
    xwp_ref[...] = t * dinv
    dinv_ref[...] = dinv


def _mm2_body(agg_ref, dinv_ref, b1_ref, w2_ref, xwp2_ref):
    dinv = dinv_ref[...]
    h = jnp.maximum(agg_ref[...] * dinv + b1_ref[...], 0.0)
    t = jnp.dot(h, w2_ref[...], preferred_element_type=jnp.float32)
    xwp2_ref[...] = t * dinv


def _final_body(agg_ref, dinv_ref, b2_ref, out_ref):
    out_ref[...] = agg_ref[...] * dinv_ref[...] + b2_ref[...]


def kernel(x, edge_index, edge_weight, W1, b1, W2, b2):
    src = edge_index[0].astype(jnp.int32)
    dst = edge_index[1].astype(jnp.int32)
    x_p = jnp.pad(x, ((0, NP_ - N), (0, 0)))
    pad = EP - E
    pad_idx = jnp.arange(pad, dtype=jnp.int32) % N
    src_p = jnp.concatenate([src, pad_idx]).reshape(NT, R, LANE)
    dst_p = jnp.concatenate([dst, pad_idx]).reshape(NT, R, LANE)
    w_p = jnp.concatenate(
        [edge_weight, jnp.zeros((pad,), jnp.float32)]).reshape(NT, R, LANE)

    deg_parts = _deg_kernel(dst_p, w_p)
    deg2 = deg_parts.reshape(NCORE, NP_, 1)

    b1r = b1.reshape(1, D)
    b2r = b2.reshape(1, D)

    xwp1, dinv = pl.pallas_call(
        _mm1_body,
        grid=(GRID,),
        in_specs=[
            pl.BlockSpec((BLK, D), lambda i: (i, 0)),
            pl.BlockSpec((D, D), lambda i: (0, 0)),
            pl.BlockSpec((NCORE, BLK, 1), lambda i: (0, i, 0)),
        ],
        out_specs=[
            pl.BlockSpec((BLK, D), lambda i: (i, 0)),
            pl.BlockSpec((BLK, 1), lambda i: (i, 0)),
        ],
        out_shape=[
            jax.ShapeDtypeStruct((NP_, D), jnp.float32),
            jax.ShapeDtypeStruct((NP_, 1), jnp.float32),
        ],
    )(x_p, W1, deg2)

    agg1 = _msg_kernel(xwp1, src_p, dst_p, w_p)

    xwp2 = pl.pallas_call(
        _mm2_body,
        grid=(GRID,),
        in_specs=[
            pl.BlockSpec((BLK, D), lambda i: (i, 0)),
            pl.BlockSpec((BLK, 1), lambda i: (i, 0)),
            pl.BlockSpec((1, D), lambda i: (0, 0)),
            pl.BlockSpec((D, D), lambda i: (0, 0)),
        ],
        out_specs=pl.BlockSpec((BLK, D), lambda i: (i, 0)),
        out_shape=jax.ShapeDtypeStruct((NP_, D), jnp.float32),
    )(agg1, dinv, b1r, W2)

    out = _msg_final_kernel(xwp2, src_p, dst_p, w_p,
                            dinv.reshape(NP_), b2)
    return out


# speedup vs baseline: 1.1938x; 1.0431x over previous
import functools

import jax
import jax.numpy as jnp
from jax import lax
from jax.experimental import pallas as pl
from jax.experimental.pallas import tpu as pltpu
from jax.experimental.pallas import tpu_sc as plsc

N = 10000
E = 320000
D = 128
H = 64
NT = 16
NCORE = 2
LANE = 128
R = 160
EP = NT * R * LANE
K = 4
KD = 8
RD = R // NCORE
NP_ = NT * 640
N_DEG = NP_
ROWS_T = NP_ // NT
BLK = 1280
GRID = NP_ // BLK

_mesh = plsc.VectorSubcoreMesh(core_axis_name="c", subcore_axis_name="s")
_sc_params = pltpu.CompilerParams(use_tc_tiling_on_sc=False)


KD = 5
NDCH = RD // KD

@functools.partial(
    pl.kernel,
    out_type=jax.ShapeDtypeStruct((NCORE, N_DEG), jnp.float32),
    mesh=_mesh,
    compiler_params=_sc_params,
    scratch_types=[
        pltpu.VMEM_SHARED((N_DEG,), jnp.float32),
        pltpu.VMEM((4, KD, LANE), jnp.int32),
        pltpu.VMEM((4, KD, LANE), jnp.float32),
        pltpu.VMEM((640,), jnp.float32),
        pltpu.SemaphoreType.DMA((4,)),
        pltpu.SemaphoreType.DMA((2,)),
    ],
)
def _deg_kernel(dst_hbm, w_hbm, deg_hbm, deg_sp, dst_v, w_v, zz, isem, ssem):
    c = lax.axis_index("c")
    s = lax.axis_index("s")
    for i in range(640 // 16):
        zz[pl.ds(i * 16, 16)] = jnp.zeros((16,), jnp.float32)

    def idx_copies(i, bi):
        r0 = c * RD + i * KD
        return [
            pltpu.make_async_copy(dst_hbm.at[s].at[pl.ds(r0, KD)],
                                  dst_v.at[bi], isem.at[bi]),
            pltpu.make_async_copy(w_hbm.at[s].at[pl.ds(r0, KD)],
                                  w_v.at[bi], isem.at[bi]),
        ]

    for cp in idx_copies(0, 0) + idx_copies(1, 1):
        cp.start()
    pltpu.sync_copy(zz, deg_sp.at[pl.ds(s * 640, 640)])
    plsc.subcore_barrier()

    def scatters(bi, br):
        return [pltpu.make_async_copy(w_v.at[bi].at[j],
                                      deg_sp.at[dst_v.at[bi].at[j]],
                                      ssem.at[br])
                for j in range(KD)]

    def outer(i2, carry):
        for b in range(4):
            i = i2 * 4 + b
            br = b % 2
            for cp in idx_copies(i, b):
                cp.wait()
            if b < 2:
                @pl.when(i2 >= 1)
                def _():
                    for cp in scatters((b + 2) % 4, br):
                        cp.wait()
            else:
                for cp in scatters((b + 2) % 4, br):
                    cp.wait()
            for j in range(KD):
                pltpu.async_copy(w_v.at[b].at[j],
                                 deg_sp.at[dst_v.at[b].at[j]],
                                 ssem.at[br], add=True)

            @pl.when(i < NDCH - 2)
            def _(i=i, b=b):
                for cp in idx_copies(i + 2, (b + 2) % 4):
                    cp.start()
        return carry

    lax.fori_loop(0, NDCH // 4, outer, 0)
    for cp in scatters(2, 0) + scatters(3, 1):
        cp.wait()
    plsc.subcore_barrier()
    pltpu.sync_copy(deg_sp.at[pl.ds(s * 640, 640)],
                    deg_hbm.at[c].at[pl.ds(s * 640, 640)])


KM = 1
NCHUNK = R // KM

def _make_msg_kernel(final):
    out_shape = (jax.ShapeDtypeStruct((N, D), jnp.float32) if final
                 else jax.ShapeDtypeStruct((NP_, D), jnp.float32))
    extra_scratch = ([pltpu.VMEM((ROWS_T,), jnp.float32),
                      pltpu.VMEM((H,), jnp.float32)] if final else [])

    def body(*refs):
        if final:
            (xwp_hbm, src_hbm, dst_hbm, w_hbm, dinv_hbm, b_hbm, out_hbm,
             xw_sp, agg_sp, src_v, dst_v, w_v, rows_v,
             gsem, ssem, isem, lsem, dinv_v, bias_v) = refs
        else:
            (xwp_hbm, src_hbm, dst_hbm, w_hbm, out_hbm,
             xw_sp, agg_sp, src_v, dst_v, w_v, rows_v,
             gsem, ssem, isem, lsem) = refs
        c = lax.axis_index("c")
        s = lax.axis_index("s")
        n0 = s * ROWS_T
        half = xwp_hbm.at[pl.ds(n0, ROWS_T), pl.ds(c * H, H)]
        loads = [
            pltpu.make_async_copy(half, xw_sp.at[pl.ds(n0, ROWS_T)], lsem),
            pltpu.make_async_copy(half, agg_sp.at[pl.ds(n0, ROWS_T)], lsem),
        ]
        if final:
            loads.append(pltpu.make_async_copy(
                dinv_hbm.at[pl.ds(n0, ROWS_T)], dinv_v, lsem))
            loads.append(pltpu.make_async_copy(
                b_hbm.at[pl.ds(c * H, H)], bias_v, lsem))
        for cp in loads:
            cp.start()

        def idx_copies(i, b4, b8):
            r0 = i * KM
            return [
                pltpu.make_async_copy(src_hbm.at[s].at[r0],
                                      src_v.at[b4], isem.at[b4]),
                pltpu.make_async_copy(dst_hbm.at[s].at[r0],
                                      dst_v.at[b8], isem.at[b4]),
                pltpu.make_async_copy(w_hbm.at[s].at[r0],
                                      w_v.at[b4], isem.at[b4]),
            ]

        for cp in idx_copies(0, 0, 0) + idx_copies(1, 1, 1):
            cp.start()

        def _stage(q, carry):
            return carry

        for cp in loads:
            cp.wait()
        plsc.subcore_barrier()

        def gather_cp(i, b4):
            return pltpu.make_async_copy(
                xw_sp.at[src_v.at[b4]], rows_v.at[b4], gsem.at[b4])

        def scatter_cp(i, b4, b8):
            return pltpu.make_async_copy(
                rows_v.at[b4], agg_sp.at[dst_v.at[b8]], ssem.at[b4])

        def scale_chunk(i, b4):
            def _scale(g):
                w16 = w_v[b4, pl.ds(g, 16)]
                for l in range(16):
                    wv = w16[l]
                    e = g + l
                    for f in range(H // 16):
                        rows_v[b4, e, pl.ds(f * 16, 16)] = (
                            rows_v[b4, e, pl.ds(f * 16, 16)] * wv)

            plsc.parallel_loop(0, LANE, step=16, unroll=2)(_scale)

        def process(i, b4, b8):
            gather_cp(i, b4).wait()
            scale_chunk(i, b4)
            pltpu.async_copy(rows_v.at[b4], agg_sp.at[dst_v.at[b8]],
                             ssem.at[b4], add=True)

        def outer(i2, carry):
            for b8 in range(8):
                i = i2 * 8 + b8
                b4 = b8 % 4
                for cp in idx_copies(i, b4, b8):
                    cp.wait()

                if b8 < 4:
                    @pl.when(i2 >= 1)
                    def _():
                        scatter_cp(i - 4, b4, (b8 + 4) % 8).wait()
                else:
                    scatter_cp(i - 4, b4, (b8 + 4) % 8).wait()

                gather_cp(i, b4).start()

                if b8 < 2:
                    @pl.when(i2 >= 1)
                    def _():
                        process(i - 2, (b8 + 2) % 4, (b8 + 6) % 8)
                else:
                    process(i - 2, (b8 + 2) % 4, (b8 + 6) % 8)

                @pl.when(i < NCHUNK - 2)
                def _(i=i, b8=b8):
                    for cp in idx_copies(i + 2, (b8 + 2) % 4, (b8 + 2) % 8):
                        cp.start()
            return carry

        lax.fori_loop(0, NCHUNK // 8, outer, 0)
        process(NCHUNK - 2, 2, 6)
        process(NCHUNK - 1, 3, 7)
        for q in range(4):
            scatter_cp(NCHUNK - 4 + q, q, (4 + q) % 8).wait()
        plsc.subcore_barrier()

        if not final:
            pltpu.sync_copy(agg_sp.at[pl.ds(n0, ROWS_T)],
                            out_hbm.at[pl.ds(n0, ROWS_T), pl.ds(c * H, H)])
            return

        b4 = [bias_v[pl.ds(f * 16, 16)] for f in range(H // 16)]
        WB = 80

        def _wb_slab(q, carry):
            off = q * WB

            @pl.when(n0 + off < N)
            def _():
                pltpu.sync_copy(agg_sp.at[pl.ds(n0 + off, WB)],
                                rows_v.at[0].at[pl.ds(0, WB)])

                def _wb(g):
                    d16 = dinv_v[pl.ds(off + g, 16)]
                    for l in range(16):
                        dv = d16[l]
                        for f in range(H // 16):
                            rows_v[0, g + l, pl.ds(f * 16, 16)] = (
                                rows_v[0, g + l, pl.ds(f * 16, 16)] * dv
                                + b4[f])

                plsc.parallel_loop(0, WB, step=16)(_wb)
                pltpu.sync_copy(
                    rows_v.at[0].at[pl.ds(0, WB)],
                    out_hbm.at[pl.ds(n0 + off, WB), pl.ds(c * H, H)])

            return carry

        lax.fori_loop(0, ROWS_T // WB, _wb_slab, 0)

    return functools.partial(
        pl.kernel,
        out_type=out_shape,
        mesh=_mesh,
        compiler_params=_sc_params,
        scratch_types=[
            pltpu.VMEM_SHARED((NP_, H), jnp.float32),
            pltpu.VMEM_SHARED((NP_, H), jnp.float32),
            pltpu.VMEM((4, LANE), jnp.int32),
            pltpu.VMEM((8, LANE), jnp.int32),
            pltpu.VMEM((4, LANE), jnp.float32),
            pltpu.VMEM((4, LANE, H), jnp.float32),
            pltpu.SemaphoreType.DMA((4,)),
            pltpu.SemaphoreType.DMA((4,)),
            pltpu.SemaphoreType.DMA((4,)),
            pltpu.SemaphoreType.DMA,
        ] + extra_scratch,
    )(body)


_msg_kernel = _make_msg_kernel(final=False)
_msg_final_kernel = _make_msg_kernel(final=True)


def _mm1_body(x_ref, w1_ref, deg_ref, xwp_ref, dinv_ref):
    deg = deg_ref[0] + deg_ref[1] + 1.0
    dinv = lax.rsqrt(jnp.maximum(deg, 1e-12))
    t = jnp.dot(x_ref[...], w1_ref[...], preferred_element_type=jnp.float32)
    xwp_ref[...] = t * dinv
    dinv_ref[...] = dinv


def _mm2_body(agg_ref, dinv_ref, b1_ref, w2_ref, xwp2_ref):
    dinv = dinv_ref[...]
    h = jnp.maximum(agg_ref[...] * dinv + b1_ref[...], 0.0)
    t = jnp.dot(h, w2_ref[...], preferred_element_type=jnp.float32)
    xwp2_ref[...] = t * dinv


def _final_body(agg_ref, dinv_ref, b2_ref, out_ref):
    out_ref[...] = agg_ref[...] * dinv_ref[...] + b2_ref[...]


def kernel(x, edge_index, edge_weight, W1, b1, W2, b2):
    src = edge_index[0].astype(jnp.int32)
    dst = edge_index[1].astype(jnp.int32)
    x_p = jnp.pad(x, ((0, NP_ - N), (0, 0)))
    pad = EP - E
    pad_idx = jnp.arange(pad, dtype=jnp.int32) % N
    src_p = jnp.concatenate([src, pad_idx]).reshape(NT, R, LANE)
    dst_p = jnp.concatenate([dst, pad_idx]).reshape(NT, R, LANE)
    w_p = jnp.concatenate(
        [edge_weight, jnp.zeros((pad,), jnp.float32)]).reshape(NT, R, LANE)

    deg_parts = _deg_kernel(dst_p, w_p)
    deg2 = deg_parts.reshape(NCORE, NP_, 1)

    b1r = b1.reshape(1, D)
    b2r = b2.reshape(1, D)

    xwp1, dinv = pl.pallas_call(
        _mm1_body,
        grid=(GRID,),
        in_specs=[
            pl.BlockSpec((BLK, D), lambda i: (i, 0)),
            pl.BlockSpec((D, D), lambda i: (0, 0)),
            pl.BlockSpec((NCORE, BLK, 1), lambda i: (0, i, 0)),
        ],
        out_specs=[
            pl.BlockSpec((BLK, D), lambda i: (i, 0)),
            pl.BlockSpec((BLK, 1), lambda i: (i, 0)),
        ],
        out_shape=[
            jax.ShapeDtypeStruct((NP_, D), jnp.float32),
            jax.ShapeDtypeStruct((NP_, 1), jnp.float32),
        ],
    )(x_p, W1, deg2)

    agg1 = _msg_kernel(xwp1, src_p, dst_p, w_p)

    xwp2 = pl.pallas_call(
        _mm2_body,
        grid=(GRID,),
        in_specs=[
            pl.BlockSpec((BLK, D), lambda i: (i, 0)),
            pl.BlockSpec((BLK, 1), lambda i: (i, 0)),
            pl.BlockSpec((1, D), lambda i: (0, 0)),
            pl.BlockSpec((D, D), lambda i: (0, 0)),
        ],
        out_specs=pl.BlockSpec((BLK, D), lambda i: (i, 0)),
        out_shape=jax.ShapeDtypeStruct((NP_, D), jnp.float32),
    )(agg1, dinv, b1r, W2)

    out = _msg_final_kernel(xwp2, src_p, dst_p, w_p,
                            dinv.reshape(NP_), b2)
    return out


# speedup vs baseline: 1.1942x; 1.0003x over previous
import functools

import jax
import jax.numpy as jnp
from jax import lax
from jax.experimental import pallas as pl
from jax.experimental.pallas import tpu as pltpu
from jax.experimental.pallas import tpu_sc as plsc

N = 10000
E = 320000
D = 128
H = 64
NT = 16
NCORE = 2
LANE = 128
R = 160
EP = NT * R * LANE
K = 4
KD = 8
RD = R // NCORE
NP_ = NT * 640
N_DEG = NP_
ROWS_T = NP_ // NT
BLK = 1280
GRID = NP_ // BLK

_mesh = plsc.VectorSubcoreMesh(core_axis_name="c", subcore_axis_name="s")
_sc_params = pltpu.CompilerParams(use_tc_tiling_on_sc=False)


KD = 5
NDCH = RD // KD

@functools.partial(
    pl.kernel,
    out_type=jax.ShapeDtypeStruct((NCORE, N_DEG), jnp.float32),
    mesh=_mesh,
    compiler_params=_sc_params,
    scratch_types=[
        pltpu.VMEM_SHARED((N_DEG,), jnp.float32),
        pltpu.VMEM((4, KD, LANE), jnp.int32),
        pltpu.VMEM((4, KD, LANE), jnp.float32),
        pltpu.VMEM((640,), jnp.float32),
        pltpu.SemaphoreType.DMA((4,)),
        pltpu.SemaphoreType.DMA((2,)),
    ],
)
def _deg_kernel(dst_hbm, w_hbm, deg_hbm, deg_sp, dst_v, w_v, zz, isem, ssem):
    c = lax.axis_index("c")
    s = lax.axis_index("s")
    for i in range(640 // 16):
        zz[pl.ds(i * 16, 16)] = jnp.zeros((16,), jnp.float32)

    def idx_copies(i, bi):
        r0 = c * RD + i * KD
        return [
            pltpu.make_async_copy(dst_hbm.at[s].at[pl.ds(r0, KD)],
                                  dst_v.at[bi], isem.at[bi]),
            pltpu.make_async_copy(w_hbm.at[s].at[pl.ds(r0, KD)],
                                  w_v.at[bi], isem.at[bi]),
        ]

    for cp in idx_copies(0, 0) + idx_copies(1, 1):
        cp.start()
    pltpu.sync_copy(zz, deg_sp.at[pl.ds(s * 640, 640)])
    plsc.subcore_barrier()

    def scatters(bi, br):
        return [pltpu.make_async_copy(w_v.at[bi].at[j],
                                      deg_sp.at[dst_v.at[bi].at[j]],
                                      ssem.at[br])
                for j in range(KD)]

    def outer(i2, carry):
        for b in range(4):
            i = i2 * 4 + b
            br = b % 2
            for cp in idx_copies(i, b):
                cp.wait()
            if b < 2:
                @pl.when(i2 >= 1)
                def _():
                    for cp in scatters((b + 2) % 4, br):
                        cp.wait()
            else:
                for cp in scatters((b + 2) % 4, br):
                    cp.wait()
            for j in range(KD):
                pltpu.async_copy(w_v.at[b].at[j],
                                 deg_sp.at[dst_v.at[b].at[j]],
                                 ssem.at[br], add=True)

            @pl.when(i < NDCH - 2)
            def _(i=i, b=b):
                for cp in idx_copies(i + 2, (b + 2) % 4):
                    cp.start()
        return carry

    lax.fori_loop(0, NDCH // 4, outer, 0)
    for cp in scatters(2, 0) + scatters(3, 1):
        cp.wait()
    plsc.subcore_barrier()
    pltpu.sync_copy(deg_sp.at[pl.ds(s * 640, 640)],
                    deg_hbm.at[c].at[pl.ds(s * 640, 640)])


KM = 1
NCHUNK = R // KM

def _make_msg_kernel(final):
    out_shape = (jax.ShapeDtypeStruct((N, D), jnp.float32) if final
                 else jax.ShapeDtypeStruct((NP_, D), jnp.float32))
    extra_scratch = ([pltpu.VMEM((ROWS_T,), jnp.float32),
                      pltpu.VMEM((H,), jnp.float32)] if final else [])

    def body(*refs):
        if final:
            (xwp_hbm, src_hbm, dst_hbm, w_hbm, dinv_hbm, b_hbm, out_hbm,
             xw_sp, agg_sp, src_v, dst_v, w_v, rows_v,
             gsem, ssem, isem, lsem, dinv_v, bias_v) = refs
        else:
            (xwp_hbm, src_hbm, dst_hbm, w_hbm, out_hbm,
             xw_sp, agg_sp, src_v, dst_v, w_v, rows_v,
             gsem, ssem, isem, lsem) = refs
        c = lax.axis_index("c")
        s = lax.axis_index("s")
        n0 = s * ROWS_T
        half = xwp_hbm.at[pl.ds(n0, ROWS_T), pl.ds(c * H, H)]
        loads = [
            pltpu.make_async_copy(half, xw_sp.at[pl.ds(n0, ROWS_T)], lsem),
            pltpu.make_async_copy(half, agg_sp.at[pl.ds(n0, ROWS_T)], lsem),
        ]
        if final:
            loads.append(pltpu.make_async_copy(
                dinv_hbm.at[pl.ds(n0, ROWS_T)], dinv_v, lsem))
            loads.append(pltpu.make_async_copy(
                b_hbm.at[pl.ds(c * H, H)], bias_v, lsem))
        for cp in loads:
            cp.start()

        def idx_copies(i, b4, b8):
            r0 = i * KM
            return [
                pltpu.make_async_copy(src_hbm.at[s].at[r0],
                                      src_v.at[b4], isem.at[b4]),
                pltpu.make_async_copy(dst_hbm.at[s].at[r0],
                                      dst_v.at[b8], isem.at[b4]),
                pltpu.make_async_copy(w_hbm.at[s].at[r0],
                                      w_v.at[b4], isem.at[b4]),
            ]

        for cp in idx_copies(0, 0, 0) + idx_copies(1, 1, 1):
            cp.start()

        for cp in loads:
            cp.wait()
        plsc.subcore_barrier()

        def gather_cp(i, b4):
            return pltpu.make_async_copy(
                xw_sp.at[src_v.at[b4]], rows_v.at[b4], gsem.at[b4])

        def scatter_cp(i, b4, b8):
            return pltpu.make_async_copy(
                rows_v.at[b4], agg_sp.at[dst_v.at[b8]], ssem.at[b4])

        def scale_chunk(i, b4):
            def _scale(g):
                w16 = w_v[b4, pl.ds(g, 16)]
                for l in range(16):
                    wv = w16[l]
                    e = g + l
                    for f in range(H // 16):
                        rows_v[b4, e, pl.ds(f * 16, 16)] = (
                            rows_v[b4, e, pl.ds(f * 16, 16)] * wv)

            plsc.parallel_loop(0, LANE, step=16, unroll=2)(_scale)

        def process(i, b4, b8):
            gather_cp(i, b4).wait()
            scale_chunk(i, b4)
            pltpu.async_copy(rows_v.at[b4], agg_sp.at[dst_v.at[b8]],
                             ssem.at[b4], add=True)

        def outer(i2, carry):
            for b8 in range(8):
                i = i2 * 8 + b8
                b4 = b8 % 4
                for cp in idx_copies(i, b4, b8):
                    cp.wait()

                if b8 < 4:
                    @pl.when(i2 >= 1)
                    def _():
                        scatter_cp(i - 4, b4, (b8 + 4) % 8).wait()
                else:
                    scatter_cp(i - 4, b4, (b8 + 4) % 8).wait()

                gather_cp(i, b4).start()

                if b8 < 2:
                    @pl.when(i2 >= 1)
                    def _():
                        process(i - 2, (b8 + 2) % 4, (b8 + 6) % 8)
                else:
                    process(i - 2, (b8 + 2) % 4, (b8 + 6) % 8)

                @pl.when(i < NCHUNK - 2)
                def _(i=i, b8=b8):
                    for cp in idx_copies(i + 2, (b8 + 2) % 4, (b8 + 2) % 8):
                        cp.start()
            return carry

        lax.fori_loop(0, NCHUNK // 8, outer, 0)
        process(NCHUNK - 2, 2, 6)
        process(NCHUNK - 1, 3, 7)
        for q in range(4):
            scatter_cp(NCHUNK - 4 + q, q, (4 + q) % 8).wait()
        plsc.subcore_barrier()

        if not final:
            pltpu.sync_copy(agg_sp.at[pl.ds(n0, ROWS_T)],
                            out_hbm.at[pl.ds(n0, ROWS_T), pl.ds(c * H, H)])
            return

        b4 = [bias_v[pl.ds(f * 16, 16)] for f in range(H // 16)]
        WB = 80

        def _wb_slab(q, carry):
            off = q * WB

            @pl.when(n0 + off < N)
            def _():
                pltpu.sync_copy(agg_sp.at[pl.ds(n0 + off, WB)],
                                rows_v.at[0].at[pl.ds(0, WB)])

                def _wb(g):
                    d16 = dinv_v[pl.ds(off + g, 16)]
                    for l in range(16):
                        dv = d16[l]
                        for f in range(H // 16):
                            rows_v[0, g + l, pl.ds(f * 16, 16)] = (
                                rows_v[0, g + l, pl.ds(f * 16, 16)] * dv
                                + b4[f])

                plsc.parallel_loop(0, WB, step=16)(_wb)
                pltpu.sync_copy(
                    rows_v.at[0].at[pl.ds(0, WB)],
                    out_hbm.at[pl.ds(n0 + off, WB), pl.ds(c * H, H)])

            return carry

        lax.fori_loop(0, ROWS_T // WB, _wb_slab, 0)

    return functools.partial(
        pl.kernel,
        out_type=out_shape,
        mesh=_mesh,
        compiler_params=_sc_params,
        scratch_types=[
            pltpu.VMEM_SHARED((NP_, H), jnp.float32),
            pltpu.VMEM_SHARED((NP_, H), jnp.float32),
            pltpu.VMEM((4, LANE), jnp.int32),
            pltpu.VMEM((8, LANE), jnp.int32),
            pltpu.VMEM((4, LANE), jnp.float32),
            pltpu.VMEM((4, LANE, H), jnp.float32),
            pltpu.SemaphoreType.DMA((4,)),
            pltpu.SemaphoreType.DMA((4,)),
            pltpu.SemaphoreType.DMA((4,)),
            pltpu.SemaphoreType.DMA,
        ] + extra_scratch,
    )(body)


_msg_kernel = _make_msg_kernel(final=False)
_msg_final_kernel = _make_msg_kernel(final=True)


def _mm1_body(x_ref, w1_ref, deg_ref, xwp_ref, dinv_ref):
    deg = deg_ref[0] + deg_ref[1] + 1.0
    dinv = lax.rsqrt(jnp.maximum(deg, 1e-12))
    t = jnp.dot(x_ref[...], w1_ref[...], preferred_element_type=jnp.float32)
    xwp_ref[...] = t * dinv
    dinv_ref[...] = dinv


def _mm2_body(agg_ref, dinv_ref, b1_ref, w2_ref, xwp2_ref):
    dinv = dinv_ref[...]
    h = jnp.maximum(agg_ref[...] * dinv + b1_ref[...], 0.0)
    t = jnp.dot(h, w2_ref[...], preferred_element_type=jnp.float32)
    xwp2_ref[...] = t * dinv


def _final_body(agg_ref, dinv_ref, b2_ref, out_ref):
    out_ref[...] = agg_ref[...] * dinv_ref[...] + b2_ref[...]


def kernel(x, edge_index, edge_weight, W1, b1, W2, b2):
    src = edge_index[0].astype(jnp.int32)
    dst = edge_index[1].astype(jnp.int32)
    x_p = jnp.pad(x, ((0, NP_ - N), (0, 0)))
    pad = EP - E
    pad_idx = jnp.arange(pad, dtype=jnp.int32) % N
    src_p = jnp.concatenate([src, pad_idx]).reshape(NT, R, LANE)
    dst_p = jnp.concatenate([dst, pad_idx]).reshape(NT, R, LANE)
    w_p = jnp.concatenate(
        [edge_weight, jnp.zeros((pad,), jnp.float32)]).reshape(NT, R, LANE)

    deg_parts = _deg_kernel(dst_p, w_p)
    deg2 = deg_parts.reshape(NCORE, NP_, 1)

    b1r = b1.reshape(1, D)
    b2r = b2.reshape(1, D)

    xwp1, dinv = pl.pallas_call(
        _mm1_body,
        grid=(GRID,),
        in_specs=[
            pl.BlockSpec((BLK, D), lambda i: (i, 0)),
            pl.BlockSpec((D, D), lambda i: (0, 0)),
            pl.BlockSpec((NCORE, BLK, 1), lambda i: (0, i, 0)),
        ],
        out_specs=[
            pl.BlockSpec((BLK, D), lambda i: (i, 0)),
            pl.BlockSpec((BLK, 1), lambda i: (i, 0)),
        ],
        out_shape=[
            jax.ShapeDtypeStruct((NP_, D), jnp.float32),
            jax.ShapeDtypeStruct((NP_, 1), jnp.float32),
        ],
    )(x_p, W1, deg2)

    agg1 = _msg_kernel(xwp1, src_p, dst_p, w_p)

    xwp2 = pl.pallas_call(
        _mm2_body,
        grid=(GRID,),
        in_specs=[
            pl.BlockSpec((BLK, D), lambda i: (i, 0)),
            pl.BlockSpec((BLK, 1), lambda i: (i, 0)),
            pl.BlockSpec((1, D), lambda i: (0, 0)),
            pl.BlockSpec((D, D), lambda i: (0, 0)),
        ],
        out_specs=pl.BlockSpec((BLK, D), lambda i: (i, 0)),
        out_shape=jax.ShapeDtypeStruct((NP_, D), jnp.float32),
    )(agg1, dinv, b1r, W2)

    out = _msg_final_kernel(xwp2, src_p, dst_p, w_p,
                            dinv.reshape(NP_), b2)
    return out


# speedup vs baseline: 1.2075x; 1.0112x over previous
import functools

import jax
import jax.numpy as jnp
from jax import lax
from jax.experimental import pallas as pl
from jax.experimental.pallas import tpu as pltpu
from jax.experimental.pallas import tpu_sc as plsc

N = 10000
E = 320000
D = 128
H = 64
NT = 16
NCORE = 2
LANE = 128
R = 160
EP = NT * R * LANE
K = 4
KD = 8
RD = R // NCORE
NP_ = NT * 640
N_DEG = NP_
ROWS_T = NP_ // NT
BLK = 2560
GRID = NP_ // BLK

_mesh = plsc.VectorSubcoreMesh(core_axis_name="c", subcore_axis_name="s")
_sc_params = pltpu.CompilerParams(use_tc_tiling_on_sc=False)


KD = 5
NDCH = RD // KD

@functools.partial(
    pl.kernel,
    out_type=jax.ShapeDtypeStruct((NCORE, N_DEG), jnp.float32),
    mesh=_mesh,
    compiler_params=_sc_params,
    scratch_types=[
        pltpu.VMEM_SHARED((N_DEG,), jnp.float32),
        pltpu.VMEM((4, KD, LANE), jnp.int32),
        pltpu.VMEM((4, KD, LANE), jnp.float32),
        pltpu.VMEM((640,), jnp.float32),
        pltpu.SemaphoreType.DMA((4,)),
        pltpu.SemaphoreType.DMA((2,)),
    ],
)
def _deg_kernel(dst_hbm, w_hbm, deg_hbm, deg_sp, dst_v, w_v, zz, isem, ssem):
    c = lax.axis_index("c")
    s = lax.axis_index("s")
    for i in range(640 // 16):
        zz[pl.ds(i * 16, 16)] = jnp.zeros((16,), jnp.float32)

    def idx_copies(i, bi):
        r0 = c * RD + i * KD
        return [
            pltpu.make_async_copy(dst_hbm.at[s].at[pl.ds(r0, KD)],
                                  dst_v.at[bi], isem.at[bi]),
            pltpu.make_async_copy(w_hbm.at[s].at[pl.ds(r0, KD)],
                                  w_v.at[bi], isem.at[bi]),
        ]

    for cp in idx_copies(0, 0) + idx_copies(1, 1):
        cp.start()
    pltpu.sync_copy(zz, deg_sp.at[pl.ds(s * 640, 640)])
    plsc.subcore_barrier()

    def scatters(bi, br):
        return [pltpu.make_async_copy(w_v.at[bi].at[j],
                                      deg_sp.at[dst_v.at[bi].at[j]],
                                      ssem.at[br])
                for j in range(KD)]

    def outer(i2, carry):
        for b in range(4):
            i = i2 * 4 + b
            br = b % 2
            for cp in idx_copies(i, b):
                cp.wait()
            if b < 2:
                @pl.when(i2 >= 1)
                def _():
                    for cp in scatters((b + 2) % 4, br):
                        cp.wait()
            else:
                for cp in scatters((b + 2) % 4, br):
                    cp.wait()
            for j in range(KD):
                pltpu.async_copy(w_v.at[b].at[j],
                                 deg_sp.at[dst_v.at[b].at[j]],
                                 ssem.at[br], add=True)

            @pl.when(i < NDCH - 2)
            def _(i=i, b=b):
                for cp in idx_copies(i + 2, (b + 2) % 4):
                    cp.start()
        return carry

    lax.fori_loop(0, NDCH // 4, outer, 0)
    for cp in scatters(2, 0) + scatters(3, 1):
        cp.wait()
    plsc.subcore_barrier()
    pltpu.sync_copy(deg_sp.at[pl.ds(s * 640, 640)],
                    deg_hbm.at[c].at[pl.ds(s * 640, 640)])


KM = 1
NCHUNK = R // KM

def _make_msg_kernel(final):
    out_shape = (jax.ShapeDtypeStruct((N, D), jnp.float32) if final
                 else jax.ShapeDtypeStruct((NP_, D), jnp.float32))
    extra_scratch = ([pltpu.VMEM((ROWS_T,), jnp.float32),
                      pltpu.VMEM((H,), jnp.float32)] if final else [])

    def body(*refs):
        if final:
            (xwp_hbm, src_hbm, dst_hbm, w_hbm, dinv_hbm, b_hbm, out_hbm,
             xw_sp, agg_sp, src_v, dst_v, w_v, rows_v,
             gsem, ssem, isem, lsem, dinv_v, bias_v) = refs
        else:
            (xwp_hbm, src_hbm, dst_hbm, w_hbm, out_hbm,
             xw_sp, agg_sp, src_v, dst_v, w_v, rows_v,
             gsem, ssem, isem, lsem) = refs
        c = lax.axis_index("c")
        s = lax.axis_index("s")
        n0 = s * ROWS_T
        half = xwp_hbm.at[pl.ds(n0, ROWS_T), pl.ds(c * H, H)]
        loads = [
            pltpu.make_async_copy(half, xw_sp.at[pl.ds(n0, ROWS_T)], lsem),
            pltpu.make_async_copy(half, agg_sp.at[pl.ds(n0, ROWS_T)], lsem),
        ]
        if final:
            loads.append(pltpu.make_async_copy(
                dinv_hbm.at[pl.ds(n0, ROWS_T)], dinv_v, lsem))
            loads.append(pltpu.make_async_copy(
                b_hbm.at[pl.ds(c * H, H)], bias_v, lsem))
        for cp in loads:
            cp.start()

        def idx_copies(i, b4, b8):
            r0 = i * KM
            return [
                pltpu.make_async_copy(src_hbm.at[s].at[r0],
                                      src_v.at[b4], isem.at[b4]),
                pltpu.make_async_copy(dst_hbm.at[s].at[r0],
                                      dst_v.at[b8], isem.at[b4]),
                pltpu.make_async_copy(w_hbm.at[s].at[r0],
                                      w_v.at[b4], isem.at[b4]),
            ]

        for cp in idx_copies(0, 0, 0) + idx_copies(1, 1, 1):
            cp.start()

        for cp in loads:
            cp.wait()
        plsc.subcore_barrier()

        def gather_cp(i, b4):
            return pltpu.make_async_copy(
                xw_sp.at[src_v.at[b4]], rows_v.at[b4], gsem.at[b4])

        def scatter_cp(i, b4, b8):
            return pltpu.make_async_copy(
                rows_v.at[b4], agg_sp.at[dst_v.at[b8]], ssem.at[b4])

        def scale_chunk(i, b4):
            def _scale(g):
                w16 = w_v[b4, pl.ds(g, 16)]
                for l in range(16):
                    wv = w16[l]
                    e = g + l
                    for f in range(H // 16):
                        rows_v[b4, e, pl.ds(f * 16, 16)] = (
                            rows_v[b4, e, pl.ds(f * 16, 16)] * wv)

            plsc.parallel_loop(0, LANE, step=16, unroll=2)(_scale)

        def process(i, b4, b8):
            gather_cp(i, b4).wait()
            scale_chunk(i, b4)
            pltpu.async_copy(rows_v.at[b4], agg_sp.at[dst_v.at[b8]],
                             ssem.at[b4], add=True)

        def outer(i2, carry):
            for b8 in range(8):
                i = i2 * 8 + b8
                b4 = b8 % 4
                for cp in idx_copies(i, b4, b8):
                    cp.wait()

                if b8 < 4:
                    @pl.when(i2 >= 1)
                    def _():
                        scatter_cp(i - 4, b4, (b8 + 4) % 8).wait()
                else:
                    scatter_cp(i - 4, b4, (b8 + 4) % 8).wait()

                gather_cp(i, b4).start()

                if b8 < 2:
                    @pl.when(i2 >= 1)
                    def _():
                        process(i - 2, (b8 + 2) % 4, (b8 + 6) % 8)
                else:
                    process(i - 2, (b8 + 2) % 4, (b8 + 6) % 8)

                @pl.when(i < NCHUNK - 2)
                def _(i=i, b8=b8):
                    for cp in idx_copies(i + 2, (b8 + 2) % 4, (b8 + 2) % 8):
                        cp.start()
            return carry

        lax.fori_loop(0, NCHUNK // 8, outer, 0)
        process(NCHUNK - 2, 2, 6)
        process(NCHUNK - 1, 3, 7)
        for q in range(4):
            scatter_cp(NCHUNK - 4 + q, q, (4 + q) % 8).wait()
        plsc.subcore_barrier()

        if not final:
            pltpu.sync_copy(agg_sp.at[pl.ds(n0, ROWS_T)],
                            out_hbm.at[pl.ds(n0, ROWS_T), pl.ds(c * H, H)])
            return

        b4 = [bias_v[pl.ds(f * 16, 16)] for f in range(H // 16)]
        WB = 80

        def _wb_slab(q, carry):
            off = q * WB

            @pl.when(n0 + off < N)
            def _():
                pltpu.sync_copy(agg_sp.at[pl.ds(n0 + off, WB)],
                                rows_v.at[0].at[pl.ds(0, WB)])

                def _wb(g):
                    d16 = dinv_v[pl.ds(off + g, 16)]
                    for l in range(16):
                        dv = d16[l]
                        for f in range(H // 16):
                            rows_v[0, g + l, pl.ds(f * 16, 16)] = (
                                rows_v[0, g + l, pl.ds(f * 16, 16)] * dv
                                + b4[f])

                plsc.parallel_loop(0, WB, step=16)(_wb)
                pltpu.sync_copy(
                    rows_v.at[0].at[pl.ds(0, WB)],
                    out_hbm.at[pl.ds(n0 + off, WB), pl.ds(c * H, H)])

            return carry

        lax.fori_loop(0, ROWS_T // WB, _wb_slab, 0)

    return functools.partial(
        pl.kernel,
        out_type=out_shape,
        mesh=_mesh,
        compiler_params=_sc_params,
        scratch_types=[
            pltpu.VMEM_SHARED((NP_, H), jnp.float32),
            pltpu.VMEM_SHARED((NP_, H), jnp.float32),
            pltpu.VMEM((4, LANE), jnp.int32),
            pltpu.VMEM((8, LANE), jnp.int32),
            pltpu.VMEM((4, LANE), jnp.float32),
            pltpu.VMEM((4, LANE, H), jnp.float32),
            pltpu.SemaphoreType.DMA((4,)),
            pltpu.SemaphoreType.DMA((4,)),
            pltpu.SemaphoreType.DMA((4,)),
            pltpu.SemaphoreType.DMA,
        ] + extra_scratch,
    )(body)


_msg_kernel = _make_msg_kernel(final=False)
_msg_final_kernel = _make_msg_kernel(final=True)


def _mm1_body(x_ref, w1_ref, deg_ref, xwp_ref, dinv_ref):
    deg = deg_ref[0] + deg_ref[1] + 1.0
    dinv = lax.rsqrt(jnp.maximum(deg, 1e-12))
    t = jnp.dot(x_ref[...], w1_ref[...], preferred_element_type=jnp.float32)
    xwp_ref[...] = t * dinv
    dinv_ref[...] = dinv


def _mm2_body(agg_ref, dinv_ref, b1_ref, w2_ref, xwp2_ref):
    dinv = dinv_ref[...]
    h = jnp.maximum(agg_ref[...] * dinv + b1_ref[...], 0.0)
    t = jnp.dot(h, w2_ref[...], preferred_element_type=jnp.float32)
    xwp2_ref[...] = t * dinv


def _final_body(agg_ref, dinv_ref, b2_ref, out_ref):
    out_ref[...] = agg_ref[...] * dinv_ref[...] + b2_ref[...]


def kernel(x, edge_index, edge_weight, W1, b1, W2, b2):
    src = edge_index[0].astype(jnp.int32)
    dst = edge_index[1].astype(jnp.int32)
    x_p = jnp.pad(x, ((0, NP_ - N), (0, 0)))
    pad = EP - E
    pad_idx = jnp.arange(pad, dtype=jnp.int32) % N
    src_p = jnp.concatenate([src, pad_idx]).reshape(NT, R, LANE)
    dst_p = jnp.concatenate([dst, pad_idx]).reshape(NT, R, LANE)
    w_p = jnp.concatenate(
        [edge_weight, jnp.zeros((pad,), jnp.float32)]).reshape(NT, R, LANE)

    deg_parts = _deg_kernel(dst_p, w_p)
    deg2 = deg_parts.reshape(NCORE, NP_, 1)

    b1r = b1.reshape(1, D)
    b2r = b2.reshape(1, D)

    xwp1, dinv = pl.pallas_call(
        _mm1_body,
        grid=(GRID,),
        in_specs=[
            pl.BlockSpec((BLK, D), lambda i: (i, 0)),
            pl.BlockSpec((D, D), lambda i: (0, 0)),
            pl.BlockSpec((NCORE, BLK, 1), lambda i: (0, i, 0)),
        ],
        out_specs=[
            pl.BlockSpec((BLK, D), lambda i: (i, 0)),
            pl.BlockSpec((BLK, 1), lambda i: (i, 0)),
        ],
        out_shape=[
            jax.ShapeDtypeStruct((NP_, D), jnp.float32),
            jax.ShapeDtypeStruct((NP_, 1), jnp.float32),
        ],
    )(x_p, W1, deg2)

    agg1 = _msg_kernel(xwp1, src_p, dst_p, w_p)

    xwp2 = pl.pallas_call(
        _mm2_body,
        grid=(GRID,),
        in_specs=[
            pl.BlockSpec((BLK, D), lambda i: (i, 0)),
            pl.BlockSpec((BLK, 1), lambda i: (i, 0)),
            pl.BlockSpec((1, D), lambda i: (0, 0)),
            pl.BlockSpec((D, D), lambda i: (0, 0)),
        ],
        out_specs=pl.BlockSpec((BLK, D), lambda i: (i, 0)),
        out_shape=jax.ShapeDtypeStruct((NP_, D), jnp.float32),
    )(agg1, dinv, b1r, W2)

    out = _msg_final_kernel(xwp2, src_p, dst_p, w_p,
                            dinv.reshape(NP_), b2)
    return out


# speedup vs baseline: 1.2136x; 1.0051x over previous
import functools

import jax
import jax.numpy as jnp
from jax import lax
from jax.experimental import pallas as pl
from jax.experimental.pallas import tpu as pltpu
from jax.experimental.pallas import tpu_sc as plsc

N = 10000
E = 320000
D = 128
H = 64
NT = 16
NCORE = 2
LANE = 128
R = 160
EP = NT * R * LANE
K = 4
KD = 8
RD = R // NCORE
NP_ = NT * 640
N_DEG = NP_
ROWS_T = NP_ // NT
BLK = 5120
GRID = NP_ // BLK

_mesh = plsc.VectorSubcoreMesh(core_axis_name="c", subcore_axis_name="s")
_sc_params = pltpu.CompilerParams(use_tc_tiling_on_sc=False)


KD = 5
NDCH = RD // KD

@functools.partial(
    pl.kernel,
    out_type=jax.ShapeDtypeStruct((NCORE, N_DEG), jnp.float32),
    mesh=_mesh,
    compiler_params=_sc_params,
    scratch_types=[
        pltpu.VMEM_SHARED((N_DEG,), jnp.float32),
        pltpu.VMEM((4, KD, LANE), jnp.int32),
        pltpu.VMEM((4, KD, LANE), jnp.float32),
        pltpu.VMEM((640,), jnp.float32),
        pltpu.SemaphoreType.DMA((4,)),
        pltpu.SemaphoreType.DMA((2,)),
    ],
)
def _deg_kernel(dst_hbm, w_hbm, deg_hbm, deg_sp, dst_v, w_v, zz, isem, ssem):
    c = lax.axis_index("c")
    s = lax.axis_index("s")
    for i in range(640 // 16):
        zz[pl.ds(i * 16, 16)] = jnp.zeros((16,), jnp.float32)

    def idx_copies(i, bi):
        r0 = c * RD + i * KD
        return [
            pltpu.make_async_copy(dst_hbm.at[s].at[pl.ds(r0, KD)],
                                  dst_v.at[bi], isem.at[bi]),
            pltpu.make_async_copy(w_hbm.at[s].at[pl.ds(r0, KD)],
                                  w_v.at[bi], isem.at[bi]),
        ]

    for cp in idx_copies(0, 0) + idx_copies(1, 1):
        cp.start()
    pltpu.sync_copy(zz, deg_sp.at[pl.ds(s * 640, 640)])
    plsc.subcore_barrier()

    def scatters(bi, br):
        return [pltpu.make_async_copy(w_v.at[bi].at[j],
                                      deg_sp.at[dst_v.at[bi].at[j]],
                                      ssem.at[br])
                for j in range(KD)]

    def outer(i2, carry):
        for b in range(4):
            i = i2 * 4 + b
            br = b % 2
            for cp in idx_copies(i, b):
                cp.wait()
            if b < 2:
                @pl.when(i2 >= 1)
                def _():
                    for cp in scatters((b + 2) % 4, br):
                        cp.wait()
            else:
                for cp in scatters((b + 2) % 4, br):
                    cp.wait()
            for j in range(KD):
                pltpu.async_copy(w_v.at[b].at[j],
                                 deg_sp.at[dst_v.at[b].at[j]],
                                 ssem.at[br], add=True)

            @pl.when(i < NDCH - 2)
            def _(i=i, b=b):
                for cp in idx_copies(i + 2, (b + 2) % 4):
                    cp.start()
        return carry

    lax.fori_loop(0, NDCH // 4, outer, 0)
    for cp in scatters(2, 0) + scatters(3, 1):
        cp.wait()
    plsc.subcore_barrier()
    pltpu.sync_copy(deg_sp.at[pl.ds(s * 640, 640)],
                    deg_hbm.at[c].at[pl.ds(s * 640, 640)])


KM = 1
NCHUNK = R // KM

def _make_msg_kernel(final):
    out_shape = (jax.ShapeDtypeStruct((N, D), jnp.float32) if final
                 else jax.ShapeDtypeStruct((NP_, D), jnp.float32))
    extra_scratch = ([pltpu.VMEM((ROWS_T,), jnp.float32),
                      pltpu.VMEM((H,), jnp.float32)] if final else [])

    def body(*refs):
        if final:
            (xwp_hbm, src_hbm, dst_hbm, w_hbm, dinv_hbm, b_hbm, out_hbm,
             xw_sp, agg_sp, src_v, dst_v, w_v, rows_v,
             gsem, ssem, isem, lsem, dinv_v, bias_v) = refs
        else:
            (xwp_hbm, src_hbm, dst_hbm, w_hbm, out_hbm,
             xw_sp, agg_sp, src_v, dst_v, w_v, rows_v,
             gsem, ssem, isem, lsem) = refs
        c = lax.axis_index("c")
        s = lax.axis_index("s")
        n0 = s * ROWS_T
        half = xwp_hbm.at[pl.ds(n0, ROWS_T), pl.ds(c * H, H)]
        loads = [
            pltpu.make_async_copy(half, xw_sp.at[pl.ds(n0, ROWS_T)], lsem),
            pltpu.make_async_copy(half, agg_sp.at[pl.ds(n0, ROWS_T)], lsem),
        ]
        if final:
            loads.append(pltpu.make_async_copy(
                dinv_hbm.at[pl.ds(n0, ROWS_T)], dinv_v, lsem))
            loads.append(pltpu.make_async_copy(
                b_hbm.at[pl.ds(c * H, H)], bias_v, lsem))
        for cp in loads:
            cp.start()

        def idx_copies(i, b4, b8):
            r0 = i * KM
            return [
                pltpu.make_async_copy(src_hbm.at[s].at[r0],
                                      src_v.at[b4], isem.at[b4]),
                pltpu.make_async_copy(dst_hbm.at[s].at[r0],
                                      dst_v.at[b8], isem.at[b4]),
                pltpu.make_async_copy(w_hbm.at[s].at[r0],
                                      w_v.at[b4], isem.at[b4]),
            ]

        for cp in idx_copies(0, 0, 0) + idx_copies(1, 1, 1):
            cp.start()

        for cp in loads:
            cp.wait()
        plsc.subcore_barrier()

        def gather_cp(i, b4):
            return pltpu.make_async_copy(
                xw_sp.at[src_v.at[b4]], rows_v.at[b4], gsem.at[b4])

        def scatter_cp(i, b4, b8):
            return pltpu.make_async_copy(
                rows_v.at[b4], agg_sp.at[dst_v.at[b8]], ssem.at[b4])

        def scale_chunk(i, b4):
            def _scale(g):
                w16 = w_v[b4, pl.ds(g, 16)]
                for l in range(16):
                    wv = w16[l]
                    e = g + l
                    for f in range(H // 16):
                        rows_v[b4, e, pl.ds(f * 16, 16)] = (
                            rows_v[b4, e, pl.ds(f * 16, 16)] * wv)

            plsc.parallel_loop(0, LANE, step=16, unroll=2)(_scale)

        def process(i, b4, b8):
            gather_cp(i, b4).wait()
            scale_chunk(i, b4)
            pltpu.async_copy(rows_v.at[b4], agg_sp.at[dst_v.at[b8]],
                             ssem.at[b4], add=True)

        def outer(i2, carry):
            for b8 in range(8):
                i = i2 * 8 + b8
                b4 = b8 % 4
                for cp in idx_copies(i, b4, b8):
                    cp.wait()

                if b8 < 4:
                    @pl.when(i2 >= 1)
                    def _():
                        scatter_cp(i - 4, b4, (b8 + 4) % 8).wait()
                else:
                    scatter_cp(i - 4, b4, (b8 + 4) % 8).wait()

                gather_cp(i, b4).start()

                if b8 < 2:
                    @pl.when(i2 >= 1)
                    def _():
                        process(i - 2, (b8 + 2) % 4, (b8 + 6) % 8)
                else:
                    process(i - 2, (b8 + 2) % 4, (b8 + 6) % 8)

                @pl.when(i < NCHUNK - 2)
                def _(i=i, b8=b8):
                    for cp in idx_copies(i + 2, (b8 + 2) % 4, (b8 + 2) % 8):
                        cp.start()
            return carry

        lax.fori_loop(0, NCHUNK // 8, outer, 0)
        process(NCHUNK - 2, 2, 6)
        process(NCHUNK - 1, 3, 7)
        for q in range(4):
            scatter_cp(NCHUNK - 4 + q, q, (4 + q) % 8).wait()
        plsc.subcore_barrier()

        if not final:
            pltpu.sync_copy(agg_sp.at[pl.ds(n0, ROWS_T)],
                            out_hbm.at[pl.ds(n0, ROWS_T), pl.ds(c * H, H)])
            return

        b4 = [bias_v[pl.ds(f * 16, 16)] for f in range(H // 16)]
        WB = 80

        def _wb_slab(q, carry):
            off = q * WB

            @pl.when(n0 + off < N)
            def _():
                pltpu.sync_copy(agg_sp.at[pl.ds(n0 + off, WB)],
                                rows_v.at[0].at[pl.ds(0, WB)])

                def _wb(g):
                    d16 = dinv_v[pl.ds(off + g, 16)]
                    for l in range(16):
                        dv = d16[l]
                        for f in range(H // 16):
                            rows_v[0, g + l, pl.ds(f * 16, 16)] = (
                                rows_v[0, g + l, pl.ds(f * 16, 16)] * dv
                                + b4[f])

                plsc.parallel_loop(0, WB, step=16)(_wb)
                pltpu.sync_copy(
                    rows_v.at[0].at[pl.ds(0, WB)],
                    out_hbm.at[pl.ds(n0 + off, WB), pl.ds(c * H, H)])

            return carry

        lax.fori_loop(0, ROWS_T // WB, _wb_slab, 0)

    return functools.partial(
        pl.kernel,
        out_type=out_shape,
        mesh=_mesh,
        compiler_params=_sc_params,
        scratch_types=[
            pltpu.VMEM_SHARED((NP_, H), jnp.float32),
            pltpu.VMEM_SHARED((NP_, H), jnp.float32),
            pltpu.VMEM((4, LANE), jnp.int32),
            pltpu.VMEM((8, LANE), jnp.int32),
            pltpu.VMEM((4, LANE), jnp.float32),
            pltpu.VMEM((4, LANE, H), jnp.float32),
            pltpu.SemaphoreType.DMA((4,)),
            pltpu.SemaphoreType.DMA((4,)),
            pltpu.SemaphoreType.DMA((4,)),
            pltpu.SemaphoreType.DMA,
        ] + extra_scratch,
    )(body)


_msg_kernel = _make_msg_kernel(final=False)
_msg_final_kernel = _make_msg_kernel(final=True)


def _mm1_body(x_ref, w1_ref, deg_ref, xwp_ref, dinv_ref):
    deg = deg_ref[0] + deg_ref[1] + 1.0
    dinv = lax.rsqrt(jnp.maximum(deg, 1e-12))
    t = jnp.dot(x_ref[...], w1_ref[...], preferred_element_type=jnp.float32)
    xwp_ref[...] = t * dinv
    dinv_ref[...] = dinv


def _mm2_body(agg_ref, dinv_ref, b1_ref, w2_ref, xwp2_ref):
    dinv = dinv_ref[...]
    h = jnp.maximum(agg_ref[...] * dinv + b1_ref[...], 0.0)
    t = jnp.dot(h, w2_ref[...], preferred_element_type=jnp.float32)
    xwp2_ref[...] = t * dinv


def _final_body(agg_ref, dinv_ref, b2_ref, out_ref):
    out_ref[...] = agg_ref[...] * dinv_ref[...] + b2_ref[...]


def kernel(x, edge_index, edge_weight, W1, b1, W2, b2):
    src = edge_index[0].astype(jnp.int32)
    dst = edge_index[1].astype(jnp.int32)
    x_p = jnp.pad(x, ((0, NP_ - N), (0, 0)))
    pad = EP - E
    pad_idx = jnp.arange(pad, dtype=jnp.int32) % N
    src_p = jnp.concatenate([src, pad_idx]).reshape(NT, R, LANE)
    dst_p = jnp.concatenate([dst, pad_idx]).reshape(NT, R, LANE)
    w_p = jnp.concatenate(
        [edge_weight, jnp.zeros((pad,), jnp.float32)]).reshape(NT, R, LANE)

    deg_parts = _deg_kernel(dst_p, w_p)
    deg2 = deg_parts.reshape(NCORE, NP_, 1)

    b1r = b1.reshape(1, D)
    b2r = b2.reshape(1, D)

    xwp1, dinv = pl.pallas_call(
        _mm1_body,
        grid=(GRID,),
        in_specs=[
            pl.BlockSpec((BLK, D), lambda i: (i, 0)),
            pl.BlockSpec((D, D), lambda i: (0, 0)),
            pl.BlockSpec((NCORE, BLK, 1), lambda i: (0, i, 0)),
        ],
        out_specs=[
            pl.BlockSpec((BLK, D), lambda i: (i, 0)),
            pl.BlockSpec((BLK, 1), lambda i: (i, 0)),
        ],
        out_shape=[
            jax.ShapeDtypeStruct((NP_, D), jnp.float32),
            jax.ShapeDtypeStruct((NP_, 1), jnp.float32),
        ],
    )(x_p, W1, deg2)

    agg1 = _msg_kernel(xwp1, src_p, dst_p, w_p)

    xwp2 = pl.pallas_call(
        _mm2_body,
        grid=(GRID,),
        in_specs=[
            pl.BlockSpec((BLK, D), lambda i: (i, 0)),
            pl.BlockSpec((BLK, 1), lambda i: (i, 0)),
            pl.BlockSpec((1, D), lambda i: (0, 0)),
            pl.BlockSpec((D, D), lambda i: (0, 0)),
        ],
        out_specs=pl.BlockSpec((BLK, D), lambda i: (i, 0)),
        out_shape=jax.ShapeDtypeStruct((NP_, D), jnp.float32),
    )(agg1, dinv, b1r, W2)

    out = _msg_final_kernel(xwp2, src_p, dst_p, w_p,
                            dinv.reshape(NP_), b2)
    return out
